# Initial kernel scaffold; baseline (speedup 1.0000x reference)
import jax
import jax.numpy as jnp
from jax.experimental import pallas as pl


def kernel(u, grid, inc):
    raise NotImplementedError("write your pallas kernel here")


# speedup vs baseline: 137.6540x; 137.6540x over previous
import functools

import jax
import jax.numpy as jnp
from jax import lax
from jax.experimental import pallas as pl
from jax.experimental.pallas import tpu as pltpu
from jax.experimental.pallas import tpu_sc as plsc

_LANES = 16
_CHUNK = 2048


def _sc_vegas_body(dim, ninc, ppw, chunk, nchunks, ncores,
                   u_hbm, grid_hbm, inc_hbm, xl_hbm, jo_hbm,
                   x_hbm, jac_hbm,
                   grid_v, inc_v, xl_v, jo_v, u_v, x_v, jac_v):
    wid = lax.axis_index("s") * ncores + lax.axis_index("c")
    base = wid * ppw
    pltpu.sync_copy(grid_hbm, grid_v)
    pltpu.sync_copy(inc_hbm, inc_v)
    pltpu.sync_copy(xl_hbm, xl_v)
    pltpu.sync_copy(jo_hbm, jo_v)
    iot = lax.iota(jnp.int32, _LANES)
    fninc = float(ninc)

    def do_group(i, _):
        flat0 = (i * _LANES + iot) * dim
        jac = jnp.full((_LANES,), 1.0, jnp.float32)
        for d in range(dim):
            ud = plsc.load_gather(u_v, [flat0 + d])
            un = ud * fninc
            iu = un.astype(jnp.int32)
            du = un - iu.astype(jnp.float32)
            msk = iu < ninc
            iuc = jnp.minimum(jnp.maximum(iu, 0), ninc - 1)
            g = plsc.load_gather(grid_v, [iuc + (d * (ninc + 1))])
            ig = plsc.load_gather(inc_v, [iuc + (d * ninc)])
            xd = jnp.where(msk, g + ig * du, xl_v[d])
            jac = jac * jnp.where(msk, ig * fninc, jo_v[d])
            plsc.store_scatter(x_v, [flat0 + d], xd)
        jac_v[pl.ds(i * _LANES, _LANES)] = jac
        return 0

    def do_chunk(k, _):
        row0 = base + k * chunk
        pltpu.sync_copy(u_hbm.at[pl.ds(row0 * dim, chunk * dim)], u_v)
        lax.fori_loop(0, chunk // _LANES, do_group, 0)
        pltpu.sync_copy(x_v, x_hbm.at[pl.ds(row0 * dim, chunk * dim)])
        pltpu.sync_copy(jac_v, jac_hbm.at[pl.ds(row0, chunk)])
        return 0

    lax.fori_loop(0, nchunks, do_chunk, 0)


def _log_body(j_ref, o_ref):
    o_ref[...] = jnp.log(j_ref[...])


def _tc_log(jacp):
    n = jacp.shape[0]
    rows = n // 128
    brows = min(2048, rows)
    j2 = jacp.reshape(rows, 128)
    out = pl.pallas_call(
        _log_body,
        grid=(rows // brows,),
        in_specs=[pl.BlockSpec((brows, 128), lambda i: (i, 0))],
        out_specs=pl.BlockSpec((brows, 128), lambda i: (i, 0)),
        out_shape=jax.ShapeDtypeStruct((rows, 128), jnp.float32),
    )(j2)
    return out.reshape(n)


def kernel(u, grid, inc):
    n, dim = u.shape
    ninc = grid.shape[1] - 1
    info = plsc.get_sparse_core_info()
    ncores, nsub = info.num_cores, info.num_subcores
    nw = ncores * nsub
    ppw = n // nw
    chunk = min(_CHUNK, ppw)
    nchunks = ppw // chunk

    xlast = jnp.broadcast_to(grid[:, -1:], (dim, _LANES))
    jout = jnp.broadcast_to(inc[:, -1:] * float(ninc), (dim, _LANES))
    grid_f = grid.reshape(dim * (ninc + 1))
    inc_f = inc.reshape(dim * ninc)

    mesh = plsc.VectorSubcoreMesh(core_axis_name="c", subcore_axis_name="s")
    run = functools.partial(
        pl.kernel,
        out_type=[
            jax.ShapeDtypeStruct((n * dim,), jnp.float32),
            jax.ShapeDtypeStruct((n,), jnp.float32),
        ],
        mesh=mesh,
        compiler_params=pltpu.CompilerParams(use_tc_tiling_on_sc=False,
                                             needs_layout_passes=False),
        scratch_types=[
            pltpu.VMEM((dim * (ninc + 1),), jnp.float32),
            pltpu.VMEM((dim * ninc,), jnp.float32),
            pltpu.VMEM((dim, _LANES), jnp.float32),
            pltpu.VMEM((dim, _LANES), jnp.float32),
            pltpu.VMEM((chunk * dim,), jnp.float32),
            pltpu.VMEM((chunk * dim,), jnp.float32),
            pltpu.VMEM((chunk,), jnp.float32),
        ],
    )(functools.partial(_sc_vegas_body, dim, ninc, ppw, chunk, nchunks,
                        ncores))

    x, jacp = run(u.reshape(n * dim), grid_f, inc_f, xlast, jout)
    return x.reshape(n, dim), _tc_log(jacp)
